# baseline (speedup 1.0000x reference)
import functools

import jax
import jax.numpy as jnp
from jax.experimental import pallas as pl
from jax.experimental.pallas import tpu as pltpu


def _pq_kernel(zt_ref, cb_ref, recont_ref, *, M, K, sub):
    for m in range(M):
        zmt = zt_ref[m * sub:(m + 1) * sub, :]
        cbm = cb_ref[m]
        dots = jnp.dot(cbm, zmt, preferred_element_type=jnp.float32)
        z2 = jnp.sum(zmt * zmt, axis=0, keepdims=True)
        c2 = jnp.sum(cbm * cbm, axis=1)[:, None]
        dist = z2 - 2.0 * dots + c2
        iota = jax.lax.broadcasted_iota(jnp.int32, dist.shape, 0)
        mn = jnp.min(dist, axis=0, keepdims=True)
        idx = jnp.min(jnp.where(dist == mn, iota, K), axis=0, keepdims=True)
        oh = (iota == idx).astype(jnp.float32)
        recont_ref[m * sub:(m + 1) * sub, :] = jnp.dot(
            cbm.T, oh, preferred_element_type=jnp.float32)


def _mlp_kernel(slab_ref, w1_ref, b1_ref, w2_ref, b2_ref, y_ref,
                out_ref, loss_ref, h_ref, *, np_, nj):
    p = pl.program_id(0)
    j = pl.program_id(1)
    part = jax.lax.dot_general(
        slab_ref[...].astype(jnp.bfloat16),
        w1_ref[...].astype(jnp.bfloat16),
        (((0,), (0,)), ((), ())),
        preferred_element_type=jnp.float32)

    @pl.when(p == 0)
    def _():
        h_ref[j] = part

    @pl.when(p != 0)
    def _():
        h_ref[j] += part

    @pl.when(p == np_ - 1)
    def _():
        h = jnp.maximum(h_ref[j] + b1_ref[...], 0.0)
        lg = jnp.dot(h.astype(jnp.bfloat16), w2_ref[...].astype(jnp.bfloat16),
                     preferred_element_type=jnp.float32)

        @pl.when(j == 0)
        def _():
            out_ref[...] = lg + b2_ref[...]

        @pl.when(j != 0)
        def _():
            out_ref[...] += lg

        @pl.when(j == nj - 1)
        def _():
            l = out_ref[...]
            Bb, Tp = l.shape
            mx = jnp.max(l, axis=1, keepdims=True)
            lse = jnp.log(jnp.sum(jnp.exp(l - mx), axis=1, keepdims=True)) + mx
            cols = jax.lax.broadcasted_iota(jnp.int32, (Bb, Tp), 1)
            ohy = (cols == y_ref[...]).astype(jnp.float32)
            ly = jnp.sum(l * ohy, axis=1, keepdims=True)
            loss_ref[...] = jnp.mean(lse - ly).reshape(1, 1)


def kernel(x_enc, y, codebook, W1, b1, W2, b2):
    B, C, H, W = x_enc.shape
    M, K, sub = codebook.shape
    P = H * W
    N = B * P
    hidden = W1.shape[1]
    tasks = W2.shape[1]
    Tp = 128

    zt = x_enc.reshape(B, C, P).transpose(1, 2, 0).reshape(C, N)

    BN = 4096
    recont = pl.pallas_call(
        lambda zr, cr, rr: _pq_kernel(zr, cr, rr, M=M, K=K, sub=sub),
        grid=(N // BN,),
        in_specs=[
            pl.BlockSpec((C, BN), lambda i: (0, i)),
            pl.BlockSpec((M, K, sub), lambda i: (0, 0, 0)),
        ],
        out_specs=pl.BlockSpec((C, BN), lambda i: (0, i)),
        out_shape=jax.ShapeDtypeStruct((C, N), jnp.float32),
    )(zt, codebook)

    W1v = W1.reshape(C, P * hidden)
    W2p = jnp.pad(W2, ((0, 0), (0, Tp - tasks)))
    b2p = jnp.pad(b2, (0, Tp - tasks), constant_values=-1e30).reshape(1, Tp)
    b1r = b1.reshape(1, hidden)
    y2 = y.astype(jnp.int32).reshape(B, 1)

    BH = 512
    nj = hidden // BH
    logits_p, loss = pl.pallas_call(
        functools.partial(_mlp_kernel, np_=P, nj=nj),
        grid=(P, nj),
        in_specs=[
            pl.BlockSpec((C, B), lambda p, j: (0, p)),
            pl.BlockSpec((C, BH), lambda p, j: (0, p * (hidden // BH) + j)),
            pl.BlockSpec((1, BH), lambda p, j: (0, j)),
            pl.BlockSpec((BH, Tp), lambda p, j: (j, 0)),
            pl.BlockSpec((1, Tp), lambda p, j: (0, 0)),
            pl.BlockSpec((B, 1), lambda p, j: (0, 0)),
        ],
        out_specs=[
            pl.BlockSpec((B, Tp), lambda p, j: (0, 0)),
            pl.BlockSpec((1, 1), lambda p, j: (0, 0)),
        ],
        out_shape=[
            jax.ShapeDtypeStruct((B, Tp), jnp.float32),
            jax.ShapeDtypeStruct((1, 1), jnp.float32),
        ],
        scratch_shapes=[pltpu.VMEM((nj, B, BH), jnp.float32)],
    )(recont, W1v, b1r, W2p, b2p, y2)

    return logits_p[:, :tasks], loss[0, 0]


# speedup vs baseline: 1.0265x; 1.0265x over previous
import functools

import jax
import jax.numpy as jnp
from jax.experimental import pallas as pl
from jax.experimental.pallas import tpu as pltpu


def _pq_kernel(zt_ref, cb_ref, recont_ref, *, M, K, sub):
    for m in range(M):
        zmt = zt_ref[m * sub:(m + 1) * sub, :]
        cbm = cb_ref[m]
        dots = jnp.dot(cbm, zmt, preferred_element_type=jnp.float32)
        z2 = jnp.sum(zmt * zmt, axis=0, keepdims=True)
        c2 = jnp.sum(cbm * cbm, axis=1)[:, None]
        dist = z2 - 2.0 * dots + c2
        iota = jax.lax.broadcasted_iota(jnp.int32, dist.shape, 0)
        mn = jnp.min(dist, axis=0, keepdims=True)
        idx = jnp.min(jnp.where(dist == mn, iota, K), axis=0, keepdims=True)
        oh = (iota == idx).astype(jnp.float32)
        recont_ref[m * sub:(m + 1) * sub, :] = jnp.dot(
            cbm.T, oh, preferred_element_type=jnp.float32)


def _mlp_kernel(slab_ref, w1_ref, b1_ref, w2_ref, b2_ref, y_ref,
                out_ref, loss_ref, h_ref, *, np_, nj):
    p = pl.program_id(0)
    j = pl.program_id(1)
    part = jax.lax.dot_general(
        slab_ref[...].astype(jnp.bfloat16),
        w1_ref[...].astype(jnp.bfloat16),
        (((0,), (0,)), ((), ())),
        preferred_element_type=jnp.float32)

    @pl.when(p == 0)
    def _():
        h_ref[j] = part

    @pl.when(p != 0)
    def _():
        h_ref[j] += part

    @pl.when(p == np_ - 1)
    def _():
        h = jnp.maximum(h_ref[j] + b1_ref[...], 0.0)
        lg = jnp.dot(h.astype(jnp.bfloat16), w2_ref[...].astype(jnp.bfloat16),
                     preferred_element_type=jnp.float32)

        @pl.when(j == 0)
        def _():
            out_ref[...] = lg + b2_ref[...]

        @pl.when(j != 0)
        def _():
            out_ref[...] += lg

        @pl.when(j == nj - 1)
        def _():
            l = out_ref[...]
            Bb, Tp = l.shape
            mx = jnp.max(l, axis=1, keepdims=True)
            lse = jnp.log(jnp.sum(jnp.exp(l - mx), axis=1, keepdims=True)) + mx
            cols = jax.lax.broadcasted_iota(jnp.int32, (Bb, Tp), 1)
            ohy = (cols == y_ref[...]).astype(jnp.float32)
            ly = jnp.sum(l * ohy, axis=1, keepdims=True)
            loss_ref[...] = jnp.mean(lse - ly).reshape(1, 1)


def kernel(x_enc, y, codebook, W1, b1, W2, b2):
    B, C, H, W = x_enc.shape
    M, K, sub = codebook.shape
    P = H * W
    N = B * P
    hidden = W1.shape[1]
    tasks = W2.shape[1]
    Tp = 128

    zt = x_enc.reshape(B, C, P).transpose(1, 2, 0).reshape(C, N)

    BN = 2048
    recont = pl.pallas_call(
        lambda zr, cr, rr: _pq_kernel(zr, cr, rr, M=M, K=K, sub=sub),
        grid=(N // BN,),
        in_specs=[
            pl.BlockSpec((C, BN), lambda i: (0, i)),
            pl.BlockSpec((M, K, sub), lambda i: (0, 0, 0)),
        ],
        out_specs=pl.BlockSpec((C, BN), lambda i: (0, i)),
        out_shape=jax.ShapeDtypeStruct((C, N), jnp.float32),
    )(zt, codebook)

    W1v = W1.reshape(C, P * hidden)
    W2p = jnp.pad(W2, ((0, 0), (0, Tp - tasks)))
    b2p = jnp.pad(b2, (0, Tp - tasks), constant_values=-1e30).reshape(1, Tp)
    b1r = b1.reshape(1, hidden)
    y2 = y.astype(jnp.int32).reshape(B, 1)

    BH = 512
    nj = hidden // BH
    logits_p, loss = pl.pallas_call(
        functools.partial(_mlp_kernel, np_=P, nj=nj),
        grid=(P, nj),
        in_specs=[
            pl.BlockSpec((C, B), lambda p, j: (0, p)),
            pl.BlockSpec((C, BH), lambda p, j: (0, p * (hidden // BH) + j)),
            pl.BlockSpec((1, BH), lambda p, j: (0, j)),
            pl.BlockSpec((BH, Tp), lambda p, j: (j, 0)),
            pl.BlockSpec((1, Tp), lambda p, j: (0, 0)),
            pl.BlockSpec((B, 1), lambda p, j: (0, 0)),
        ],
        out_specs=[
            pl.BlockSpec((B, Tp), lambda p, j: (0, 0)),
            pl.BlockSpec((1, 1), lambda p, j: (0, 0)),
        ],
        out_shape=[
            jax.ShapeDtypeStruct((B, Tp), jnp.float32),
            jax.ShapeDtypeStruct((1, 1), jnp.float32),
        ],
        scratch_shapes=[pltpu.VMEM((nj, B, BH), jnp.float32)],
    )(recont, W1v, b1r, W2p, b2p, y2)

    return logits_p[:, :tasks], loss[0, 0]
